# Initial kernel scaffold; baseline (speedup 1.0000x reference)
import jax
import jax.numpy as jnp
from jax.experimental import pallas as pl


def kernel(x, conv1_w, conv1_b, conv2_w, conv2_b, w_gate):
    raise NotImplementedError("write your pallas kernel here")


# speedup vs baseline: 6.1488x; 6.1488x over previous
import jax
import jax.numpy as jnp
from jax.experimental import pallas as pl
from jax.experimental.pallas import tpu as pltpu

_LIST_DIM = [64, 64, 64, 64]
_E = 8
_K = 4
_OC = 128
_KS = 3


def _cv2(v):
    n = v.shape[0]
    mean = jnp.sum(v) / n
    var = jnp.sum((v - mean) ** 2) / (n - 1)
    return var / (mean * mean + 1e-10)


def _gate_kernel(gl_ref, wg_ref, gates_ref, loss_ref):
    S = gl_ref.shape[0]
    Bsz = gl_ref.shape[1]
    E = _E
    loss = jnp.float32(0.0)
    iota = jax.lax.broadcasted_iota(jnp.int32, (Bsz, E), 1)
    for i in range(S):
        logits = jnp.dot(gl_ref[i], wg_ref[i],
                         preferred_element_type=jnp.float32)
        m = jnp.max(logits, axis=1, keepdims=True)
        ex = jnp.exp(logits - m)
        sm = ex / jnp.sum(ex, axis=1, keepdims=True)
        remaining = sm
        mask = jnp.zeros((Bsz, E), jnp.bool_)
        for _ in range(_K):
            rowmax = jnp.max(remaining, axis=1, keepdims=True)
            ismax = remaining == rowmax
            first = jnp.min(jnp.where(ismax, iota, E), axis=1, keepdims=True)
            sel = iota == first
            mask = jnp.logical_or(mask, sel)
            remaining = jnp.where(sel, -1.0, remaining)
        kept = jnp.where(mask, sm, 0.0)
        topsum = jnp.sum(kept, axis=1, keepdims=True)
        gates = kept / (topsum + 1e-6)
        gates_ref[i] = gates
        importance = jnp.sum(gates, axis=0)
        load = jnp.sum((gates > 0).astype(jnp.float32), axis=0)
        loss = loss + _cv2(importance) + _cv2(load)
    loss_ref[:, :] = jnp.reshape(loss * 0.01, (1, 1))


def _main_kernel(gates_ref, x_ref, w1_ref, b1_ref, w2_ref, b2_ref, out_ref):
    i = pl.program_id(0)
    b = pl.program_id(1)
    dim = x_ref.shape[1]
    L = x_ref.shape[2]
    Lp = L - _KS + 1
    x = x_ref[0]
    xcat = jnp.concatenate([x[:, k:k + Lp] for k in range(_KS)], axis=0)
    h = jnp.tanh(jnp.dot(w1_ref[0], xcat,
                         preferred_element_type=jnp.float32) + b1_ref[0])
    weff = jnp.zeros((_OC, _OC), jnp.float32)
    beff = jnp.zeros((_OC, 1), jnp.float32)
    for e in range(_E):
        g = gates_ref[i, b, e]
        weff = weff + g * w2_ref[0, e]
        beff = beff + g * b2_ref[0, e]
    out_ref[0] = jnp.dot(weff, h, preferred_element_type=jnp.float32) + beff


def kernel(x, conv1_w, conv1_b, conv2_w, conv2_b, w_gate):
    B, D, L = x.shape
    S = len(_LIST_DIM)
    dim = _LIST_DIM[0]
    OC, E, KS = _OC, _E, _KS
    Lp = L - KS + 1

    gl = x[:, :, L - 6:L - 1].reshape(B, S, dim * 5).transpose(1, 0, 2)

    gates, loss2d = pl.pallas_call(
        _gate_kernel,
        out_shape=[
            jax.ShapeDtypeStruct((S, B, E), jnp.float32),
            jax.ShapeDtypeStruct((1, 1), jnp.float32),
        ],
    )(gl, w_gate)

    w1cat = jnp.transpose(conv1_w, (0, 1, 3, 2)).reshape(S, OC, KS * dim)
    b1c = conv1_b[:, :, None]
    w2r = jnp.transpose(conv2_w[:, :, :, 0].reshape(S, OC, E, OC),
                        (0, 2, 1, 3))
    b2c = jnp.transpose(conv2_b.reshape(S, OC, E), (0, 2, 1))[:, :, :, None]

    out = pl.pallas_call(
        _main_kernel,
        grid=(S, B),
        in_specs=[
            pl.BlockSpec(memory_space=pltpu.SMEM),
            pl.BlockSpec((1, dim, L), lambda i, b: (b, i, 0)),
            pl.BlockSpec((1, OC, KS * dim), lambda i, b: (i, 0, 0)),
            pl.BlockSpec((1, OC, 1), lambda i, b: (i, 0, 0)),
            pl.BlockSpec((1, E, OC, OC), lambda i, b: (i, 0, 0, 0)),
            pl.BlockSpec((1, E, OC, 1), lambda i, b: (i, 0, 0, 0)),
        ],
        out_specs=pl.BlockSpec((1, OC, Lp), lambda i, b: (b, i, 0)),
        out_shape=jax.ShapeDtypeStruct((B, S * OC, Lp), jnp.float32),
        compiler_params=pltpu.CompilerParams(
            dimension_semantics=("parallel", "parallel")),
    )(gates, x, w1cat, b1c, w2r, b2c)

    return out, loss2d[0, 0]


# speedup vs baseline: 7.4506x; 1.2117x over previous
import jax
import jax.numpy as jnp
from jax.experimental import pallas as pl
from jax.experimental.pallas import tpu as pltpu

_LIST_DIM = [64, 64, 64, 64]
_E = 8
_K = 4
_OC = 128
_KS = 3


def _cv2(v):
    n = v.shape[0]
    mean = jnp.sum(v) / n
    var = jnp.sum((v - mean) ** 2) / (n - 1)
    return var / (mean * mean + 1e-10)


def _gate_kernel(gl_ref, wg_ref, gates_ref, loss_ref):
    S = gl_ref.shape[0]
    Bsz = gl_ref.shape[1]
    E = _E
    loss = jnp.float32(0.0)
    iota = jax.lax.broadcasted_iota(jnp.int32, (Bsz, E), 1)
    for i in range(S):
        logits = jnp.dot(gl_ref[i], wg_ref[i],
                         preferred_element_type=jnp.float32)
        m = jnp.max(logits, axis=1, keepdims=True)
        ex = jnp.exp(logits - m)
        sm = ex / jnp.sum(ex, axis=1, keepdims=True)
        remaining = sm
        mask = jnp.zeros((Bsz, E), jnp.bool_)
        for _ in range(_K):
            rowmax = jnp.max(remaining, axis=1, keepdims=True)
            ismax = remaining == rowmax
            first = jnp.min(jnp.where(ismax, iota, E), axis=1, keepdims=True)
            sel = iota == first
            mask = jnp.logical_or(mask, sel)
            remaining = jnp.where(sel, -1.0, remaining)
        kept = jnp.where(mask, sm, 0.0)
        topsum = jnp.sum(kept, axis=1, keepdims=True)
        gates = kept / (topsum + 1e-6)
        gates_ref[i] = gates
        importance = jnp.sum(gates, axis=0)
        load = jnp.sum((gates > 0).astype(jnp.float32), axis=0)
        loss = loss + _cv2(importance) + _cv2(load)
    loss_ref[:, :] = jnp.reshape(loss * 0.01, (1, 1))


_BB = 4


def _main_kernel(gates_ref, x_ref, w1_ref, b1_ref, w2_ref, b2_ref, out_ref):
    i = pl.program_id(0)
    n = pl.program_id(1)
    L = x_ref.shape[2]
    Lp = L - _KS + 1
    for bb in range(_BB):
        b = n * _BB + bb
        x = x_ref[bb]
        xcat = jnp.concatenate([x[:, k:k + Lp] for k in range(_KS)], axis=0)
        h = jnp.tanh(jnp.dot(w1_ref[0], xcat,
                             preferred_element_type=jnp.float32) + b1_ref[0])
        weff = jnp.zeros((_OC, _OC), jnp.float32)
        beff = jnp.zeros((_OC, 1), jnp.float32)
        for e in range(_E):
            g = gates_ref[i, b, e]
            weff = weff + g * w2_ref[0, e]
            beff = beff + g * b2_ref[0, e]
        out_ref[bb] = jnp.dot(weff, h,
                              preferred_element_type=jnp.float32) + beff


def kernel(x, conv1_w, conv1_b, conv2_w, conv2_b, w_gate):
    B, D, L = x.shape
    S = len(_LIST_DIM)
    dim = _LIST_DIM[0]
    OC, E, KS = _OC, _E, _KS
    Lp = L - KS + 1

    gl = x[:, :, L - 6:L - 1].reshape(B, S, dim * 5).transpose(1, 0, 2)

    gates, loss2d = pl.pallas_call(
        _gate_kernel,
        out_shape=[
            jax.ShapeDtypeStruct((S, B, E), jnp.float32),
            jax.ShapeDtypeStruct((1, 1), jnp.float32),
        ],
    )(gl, w_gate)

    w1cat = jnp.transpose(conv1_w, (0, 1, 3, 2)).reshape(S, OC, KS * dim)
    b1c = conv1_b[:, :, None]
    w2r = jnp.transpose(conv2_w[:, :, :, 0].reshape(S, OC, E, OC),
                        (0, 2, 1, 3))
    b2c = jnp.transpose(conv2_b.reshape(S, OC, E), (0, 2, 1))[:, :, :, None]

    out = pl.pallas_call(
        _main_kernel,
        grid=(S, B // _BB),
        in_specs=[
            pl.BlockSpec(memory_space=pltpu.SMEM),
            pl.BlockSpec((_BB, dim, L), lambda i, n: (n, i, 0)),
            pl.BlockSpec((1, OC, KS * dim), lambda i, n: (i, 0, 0)),
            pl.BlockSpec((1, OC, 1), lambda i, n: (i, 0, 0)),
            pl.BlockSpec((1, E, OC, OC), lambda i, n: (i, 0, 0, 0)),
            pl.BlockSpec((1, E, OC, 1), lambda i, n: (i, 0, 0, 0)),
        ],
        out_specs=pl.BlockSpec((_BB, OC, Lp), lambda i, n: (n, i, 0)),
        out_shape=jax.ShapeDtypeStruct((B, S * OC, Lp), jnp.float32),
        compiler_params=pltpu.CompilerParams(
            dimension_semantics=("parallel", "parallel")),
    )(gates, x, w1cat, b1c, w2r, b2c)

    return out, loss2d[0, 0]


# speedup vs baseline: 7.6715x; 1.0297x over previous
import jax
import jax.numpy as jnp
from jax.experimental import pallas as pl
from jax.experimental.pallas import tpu as pltpu

_LIST_DIM = [64, 64, 64, 64]
_E = 8
_K = 4
_OC = 128
_KS = 3


def _cv2(v):
    n = v.shape[0]
    mean = jnp.sum(v) / n
    var = jnp.sum((v - mean) ** 2) / (n - 1)
    return var / (mean * mean + 1e-10)


def _gate_kernel(gl_ref, wg_ref, gates_ref, loss_ref):
    S = gl_ref.shape[0]
    Bsz = gl_ref.shape[1]
    E = _E
    loss = jnp.float32(0.0)
    iota = jax.lax.broadcasted_iota(jnp.int32, (Bsz, E), 1)
    for i in range(S):
        logits = jnp.dot(gl_ref[i], wg_ref[i],
                         preferred_element_type=jnp.float32)
        m = jnp.max(logits, axis=1, keepdims=True)
        ex = jnp.exp(logits - m)
        sm = ex / jnp.sum(ex, axis=1, keepdims=True)
        remaining = sm
        mask = jnp.zeros((Bsz, E), jnp.bool_)
        for _ in range(_K):
            rowmax = jnp.max(remaining, axis=1, keepdims=True)
            ismax = remaining == rowmax
            first = jnp.min(jnp.where(ismax, iota, E), axis=1, keepdims=True)
            sel = iota == first
            mask = jnp.logical_or(mask, sel)
            remaining = jnp.where(sel, -1.0, remaining)
        kept = jnp.where(mask, sm, 0.0)
        topsum = jnp.sum(kept, axis=1, keepdims=True)
        gates = kept / (topsum + 1e-6)
        gates_ref[i] = gates
        importance = jnp.sum(gates, axis=0)
        load = jnp.sum((gates > 0).astype(jnp.float32), axis=0)
        loss = loss + _cv2(importance) + _cv2(load)
    loss_ref[:, :] = jnp.reshape(loss * 0.01, (1, 1))


_BB = 8


def _main_kernel(gates_ref, x_ref, w1_ref, b1_ref, w2_ref, b2_ref, out_ref):
    i = pl.program_id(0)
    n = pl.program_id(1)
    L = x_ref.shape[2]
    Lp = L - _KS + 1
    for bb in range(_BB):
        b = n * _BB + bb
        x = x_ref[bb]
        xcat = jnp.concatenate([x[:, k:k + Lp] for k in range(_KS)], axis=0)
        h = jnp.tanh(jnp.dot(w1_ref[0], xcat,
                             preferred_element_type=jnp.float32) + b1_ref[0])
        weff = jnp.zeros((_OC, _OC), jnp.float32)
        beff = jnp.zeros((_OC, 1), jnp.float32)
        for e in range(_E):
            g = gates_ref[i, b, e]
            weff = weff + g * w2_ref[0, e]
            beff = beff + g * b2_ref[0, e]
        out_ref[bb] = jnp.dot(weff, h,
                              preferred_element_type=jnp.float32) + beff


def kernel(x, conv1_w, conv1_b, conv2_w, conv2_b, w_gate):
    B, D, L = x.shape
    S = len(_LIST_DIM)
    dim = _LIST_DIM[0]
    OC, E, KS = _OC, _E, _KS
    Lp = L - KS + 1

    gl = x[:, :, L - 6:L - 1].reshape(B, S, dim * 5).transpose(1, 0, 2)

    gates, loss2d = pl.pallas_call(
        _gate_kernel,
        out_shape=[
            jax.ShapeDtypeStruct((S, B, E), jnp.float32),
            jax.ShapeDtypeStruct((1, 1), jnp.float32),
        ],
    )(gl, w_gate)

    w1cat = jnp.transpose(conv1_w, (0, 1, 3, 2)).reshape(S, OC, KS * dim)
    b1c = conv1_b[:, :, None]
    w2r = jnp.transpose(conv2_w[:, :, :, 0].reshape(S, OC, E, OC),
                        (0, 2, 1, 3))
    b2c = jnp.transpose(conv2_b.reshape(S, OC, E), (0, 2, 1))[:, :, :, None]

    out = pl.pallas_call(
        _main_kernel,
        grid=(S, B // _BB),
        in_specs=[
            pl.BlockSpec(memory_space=pltpu.SMEM),
            pl.BlockSpec((_BB, dim, L), lambda i, n: (n, i, 0)),
            pl.BlockSpec((1, OC, KS * dim), lambda i, n: (i, 0, 0)),
            pl.BlockSpec((1, OC, 1), lambda i, n: (i, 0, 0)),
            pl.BlockSpec((1, E, OC, OC), lambda i, n: (i, 0, 0, 0)),
            pl.BlockSpec((1, E, OC, 1), lambda i, n: (i, 0, 0, 0)),
        ],
        out_specs=pl.BlockSpec((_BB, OC, Lp), lambda i, n: (n, i, 0)),
        out_shape=jax.ShapeDtypeStruct((B, S * OC, Lp), jnp.float32),
        compiler_params=pltpu.CompilerParams(
            dimension_semantics=("parallel", "parallel")),
    )(gates, x, w1cat, b1c, w2r, b2c)

    return out, loss2d[0, 0]


# speedup vs baseline: 7.7066x; 1.0046x over previous
import jax
import jax.numpy as jnp
from jax.experimental import pallas as pl
from jax.experimental.pallas import tpu as pltpu

_LIST_DIM = [64, 64, 64, 64]
_E = 8
_K = 4
_OC = 128
_KS = 3


def _cv2(v):
    n = v.shape[0]
    mean = jnp.sum(v) / n
    var = jnp.sum((v - mean) ** 2) / (n - 1)
    return var / (mean * mean + 1e-10)


def _gate_kernel(gl_ref, wg_ref, gates_ref, loss_ref):
    S = gl_ref.shape[0]
    Bsz = gl_ref.shape[1]
    E = _E
    loss = jnp.float32(0.0)
    iota = jax.lax.broadcasted_iota(jnp.int32, (Bsz, E), 1)
    for i in range(S):
        logits = jnp.dot(gl_ref[i], wg_ref[i],
                         preferred_element_type=jnp.float32)
        m = jnp.max(logits, axis=1, keepdims=True)
        ex = jnp.exp(logits - m)
        sm = ex / jnp.sum(ex, axis=1, keepdims=True)
        remaining = sm
        mask = jnp.zeros((Bsz, E), jnp.bool_)
        for _ in range(_K):
            rowmax = jnp.max(remaining, axis=1, keepdims=True)
            ismax = remaining == rowmax
            first = jnp.min(jnp.where(ismax, iota, E), axis=1, keepdims=True)
            sel = iota == first
            mask = jnp.logical_or(mask, sel)
            remaining = jnp.where(sel, -1.0, remaining)
        kept = jnp.where(mask, sm, 0.0)
        topsum = jnp.sum(kept, axis=1, keepdims=True)
        gates = kept / (topsum + 1e-6)
        gates_ref[i] = gates
        importance = jnp.sum(gates, axis=0)
        load = jnp.sum((gates > 0).astype(jnp.float32), axis=0)
        loss = loss + _cv2(importance) + _cv2(load)
    loss_ref[:, :] = jnp.reshape(loss * 0.01, (1, 1))


_BB = 8


def _main_kernel(gates_ref, x_ref, w1_ref, b1_ref, w2_ref, b2_ref, out_ref):
    i = pl.program_id(0)
    n = pl.program_id(1)
    L = x_ref.shape[2]
    Lp = L - _KS + 1
    for bb in range(_BB):
        b = n * _BB + bb
        x = x_ref[bb].astype(jnp.bfloat16)
        xcat = jnp.concatenate([x[:, k:k + Lp] for k in range(_KS)], axis=0)
        h = jnp.tanh(jnp.dot(w1_ref[0], xcat,
                             preferred_element_type=jnp.float32) + b1_ref[0])
        h = h.astype(jnp.bfloat16)
        weff = jnp.zeros((_OC, _OC), jnp.float32)
        beff = jnp.zeros((_OC, 1), jnp.float32)
        for e in range(_E):
            g = gates_ref[i, b, e]
            weff = weff + g * w2_ref[0, e]
            beff = beff + g * b2_ref[0, e]
        out_ref[bb] = jnp.dot(weff.astype(jnp.bfloat16), h,
                              preferred_element_type=jnp.float32) + beff


def kernel(x, conv1_w, conv1_b, conv2_w, conv2_b, w_gate):
    B, D, L = x.shape
    S = len(_LIST_DIM)
    dim = _LIST_DIM[0]
    OC, E, KS = _OC, _E, _KS
    Lp = L - KS + 1

    gl = x[:, :, L - 6:L - 1].reshape(B, S, dim * 5).transpose(1, 0, 2)

    gates, loss2d = pl.pallas_call(
        _gate_kernel,
        out_shape=[
            jax.ShapeDtypeStruct((S, B, E), jnp.float32),
            jax.ShapeDtypeStruct((1, 1), jnp.float32),
        ],
    )(gl, w_gate)

    w1cat = jnp.transpose(conv1_w, (0, 1, 3, 2)).reshape(
        S, OC, KS * dim).astype(jnp.bfloat16)
    b1c = conv1_b[:, :, None]
    w2r = jnp.transpose(conv2_w[:, :, :, 0].reshape(S, OC, E, OC),
                        (0, 2, 1, 3))
    b2c = jnp.transpose(conv2_b.reshape(S, OC, E), (0, 2, 1))[:, :, :, None]

    out = pl.pallas_call(
        _main_kernel,
        grid=(S, B // _BB),
        in_specs=[
            pl.BlockSpec(memory_space=pltpu.SMEM),
            pl.BlockSpec((_BB, dim, L), lambda i, n: (n, i, 0)),
            pl.BlockSpec((1, OC, KS * dim), lambda i, n: (i, 0, 0)),
            pl.BlockSpec((1, OC, 1), lambda i, n: (i, 0, 0)),
            pl.BlockSpec((1, E, OC, OC), lambda i, n: (i, 0, 0, 0)),
            pl.BlockSpec((1, E, OC, 1), lambda i, n: (i, 0, 0, 0)),
        ],
        out_specs=pl.BlockSpec((_BB, OC, Lp), lambda i, n: (n, i, 0)),
        out_shape=jax.ShapeDtypeStruct((B, S * OC, Lp), jnp.float32),
        compiler_params=pltpu.CompilerParams(
            dimension_semantics=("parallel", "parallel")),
    )(gates, x, w1cat, b1c, w2r, b2c)

    return out, loss2d[0, 0]
